# baseline (speedup 1.0000x reference)
import functools

import jax
import jax.numpy as jnp
from jax import lax
from jax.experimental import pallas as pl
from jax.experimental.pallas import tpu as pltpu
from jax.experimental.pallas import tpu_sc as plsc

N_NODES = 10000
N_PAD = 10240
N_EDGES = 320000
D = 128
N_GRAPHS = 128

NC = 2
NS = 16
NW = NC * NS
CK = 128
CB = 8
CH0 = 112
CH1 = 48
E_PAD = NS * (CH0 + CH1) * CK
E_SC0 = NS * CH0 * CK
RPT = N_PAD // NS

RB = 1000
GRID = N_NODES // RB

_mesh = plsc.VectorSubcoreMesh(core_axis_name="c", subcore_axis_name="s")


@functools.partial(
    pl.kernel,
    out_type=jax.ShapeDtypeStruct((NC, N_PAD), jnp.float32),
    mesh=_mesh,
    scratch_types=[
        pltpu.VMEM((CB, CK), jnp.int32),
        pltpu.VMEM((CK,), jnp.float32),
        pltpu.VMEM((RPT,), jnp.float32),
        pltpu.VMEM_SHARED((N_PAD,), jnp.float32),
    ],
)
def _sc_count(col_hbm, out_hbm, col_v, ones_v, zbuf, acc):
    c = lax.axis_index("c")
    s = lax.axis_index("s")
    w = c * NS + s
    for l in range(CK // 16):
        ones_v[pl.ds(l * 16, 16)] = jnp.ones((16,), jnp.float32)

    def _zb(i, carry):
        zbuf[pl.ds(i * 16, 16)] = jnp.zeros((16,), jnp.float32)
        return carry

    lax.fori_loop(0, RPT // 16, _zb, 0)
    pltpu.sync_copy(zbuf, acc.at[pl.ds(s * RPT, RPT)])
    plsc.subcore_barrier()

    def _blk(b, carry):
        pltpu.sync_copy(col_hbm.at[w, pl.ds(b * CB, CB)], col_v)

        def _body(k, carry2):
            pltpu.sync_copy(ones_v, acc.at[col_v.at[k]], add=True)
            return carry2

        lax.fori_loop(0, CB, _body, 0)
        return carry

    lax.fori_loop(0, CH0 // CB, _blk, 0)
    plsc.subcore_barrier()
    pltpu.sync_copy(acc.at[pl.ds(s * RPT, RPT)],
                    out_hbm.at[c, pl.ds(s * RPT, RPT)])


@functools.partial(
    pl.kernel,
    out_type=jax.ShapeDtypeStruct((NC, N_PAD, D), jnp.float32),
    mesh=_mesh,
    scratch_types=[
        pltpu.VMEM((CH0, CK), jnp.int32),
        pltpu.VMEM((CB, CK), jnp.int32),
        pltpu.VMEM((CK, D), jnp.float32),
        pltpu.VMEM((CK, D), jnp.float32),
        pltpu.VMEM_SHARED((N_PAD, D), jnp.float32),
        pltpu.SemaphoreType.DMA,
        pltpu.SemaphoreType.DMA,
    ],
)
def _sc_agg(y_hbm, row_hbm, col_hbm, out_hbm, row_v, col_v, g0, g1, acc,
            sem0, sem1):
    c = lax.axis_index("c")
    s = lax.axis_index("s")
    w = c * NS + s

    def _zr(i, carry):
        for l in range(D // 16):
            g0[i, pl.ds(l * 16, 16)] = jnp.zeros((16,), jnp.float32)
        return carry

    lax.fori_loop(0, CK, _zr, 0)
    for k in range(RPT // CK):
        pltpu.sync_copy(g0, acc.at[pl.ds(s * RPT + k * CK, CK)])
    plsc.subcore_barrier()
    pltpu.sync_copy(row_hbm.at[w], row_v)

    @pl.when(c == 0)
    def _():
        pltpu.async_copy(y_hbm.at[row_v.at[0]], g0, sem0)

        def _blk(b, carry):
            pltpu.sync_copy(col_hbm.at[w, pl.ds(b * CB, CB)], col_v)

            def _body(k2, carry2):
                j = b * CB + k2 * 2
                jj = k2 * 2
                pltpu.async_copy(y_hbm.at[row_v.at[j + 1]], g1, sem1)
                pltpu.make_async_copy(y_hbm.at[row_v.at[j]], g0, sem0).wait()
                pltpu.sync_copy(g0, acc.at[col_v.at[jj]], add=True)

                @pl.when(j + 2 < CH0)
                def _():
                    pltpu.async_copy(y_hbm.at[row_v.at[j + 2]], g0, sem0)

                pltpu.make_async_copy(y_hbm.at[row_v.at[j + 1]], g1, sem1).wait()
                pltpu.sync_copy(g1, acc.at[col_v.at[jj + 1]], add=True)
                return carry2

            lax.fori_loop(0, CB // 2, _body, 0)
            return carry

        lax.fori_loop(0, CH0 // CB, _blk, 0)

    @pl.when(c == 1)
    def _():
        def _blk(b, carry):
            pltpu.sync_copy(col_hbm.at[w, pl.ds(b * CB, CB)], col_v)

            def _body(k, carry2):
                j = b * CB + k
                pltpu.async_copy(y_hbm.at[row_v.at[j]], g0, sem0).wait()
                pltpu.sync_copy(g0, acc.at[col_v.at[k]], add=True)
                return carry2

            lax.fori_loop(0, CB, _body, 0)
            return carry

        lax.fori_loop(0, CH1 // CB, _blk, 0)

    plsc.subcore_barrier()
    pltpu.sync_copy(acc.at[pl.ds(s * RPT, RPT)],
                    out_hbm.at[c, pl.ds(s * RPT, RPT)])


def _tc1_body(x_ref, c0_ref, c1_ref, w1_ref, y_ref, dis_ref):
    deg = c0_ref[...] + c1_ref[...] + 1.0
    dis = lax.rsqrt(deg)
    xw = jnp.dot(x_ref[...], w1_ref[...], preferred_element_type=jnp.float32)
    y_ref[...] = xw * dis
    dis_ref[...] = dis


def _tc2_body(p0_ref, p1_ref, y1_ref, dis_ref, b1_ref, w2_ref, y2_ref):
    dis = dis_ref[...]
    h1 = jnp.maximum(dis * (p0_ref[...] + p1_ref[...] + y1_ref[...]) + b1_ref[...], 0.0)
    y2_ref[...] = jnp.dot(h1, w2_ref[...], preferred_element_type=jnp.float32) * dis


def _tc3_body(q0_ref, q1_ref, y2_ref, dis_ref, b2_ref, batch_ref, h_ref, hs_ref):
    i = pl.program_id(0)
    h = dis_ref[...] * (q0_ref[...] + q1_ref[...] + y2_ref[...]) + b2_ref[...]
    h_ref[...] = h
    gids = lax.broadcasted_iota(jnp.int32, (RB, N_GRAPHS), 1)
    oh = (batch_ref[...] == gids).astype(jnp.float32)
    contrib = lax.dot_general(oh, h, (((0,), (0,)), ((), ())),
                              preferred_element_type=jnp.float32)

    @pl.when(i == 0)
    def _():
        hs_ref[...] = contrib

    @pl.when(i != 0)
    def _():
        hs_ref[...] += contrib


_row_spec = pl.BlockSpec((RB, D), lambda i: (i, 0))
_col1_spec = pl.BlockSpec((RB, 1), lambda i: (i, 0))
_w_spec = pl.BlockSpec((D, D), lambda i: (0, 0))
_b_spec = pl.BlockSpec((1, D), lambda i: (0, 0))

_tc1 = pl.pallas_call(
    _tc1_body,
    grid=(GRID,),
    in_specs=[_row_spec, _col1_spec, _col1_spec, _w_spec],
    out_specs=[_row_spec, _col1_spec],
    out_shape=[jax.ShapeDtypeStruct((N_NODES, D), jnp.float32),
               jax.ShapeDtypeStruct((N_NODES, 1), jnp.float32)],
)

_tc2 = pl.pallas_call(
    _tc2_body,
    grid=(GRID,),
    in_specs=[_row_spec, _row_spec, _row_spec, _col1_spec, _b_spec, _w_spec],
    out_specs=_row_spec,
    out_shape=jax.ShapeDtypeStruct((N_NODES, D), jnp.float32),
)

_tc3 = pl.pallas_call(
    _tc3_body,
    grid=(GRID,),
    in_specs=[_row_spec, _row_spec, _row_spec, _col1_spec, _b_spec, _col1_spec],
    out_specs=[_row_spec, pl.BlockSpec((N_GRAPHS, D), lambda i: (0, 0))],
    out_shape=[jax.ShapeDtypeStruct((N_NODES, D), jnp.float32),
               jax.ShapeDtypeStruct((N_GRAPHS, D), jnp.float32)],
)


def kernel(x, edge_index, batch, W1, b1, W2, b2):
    row = edge_index[0].astype(jnp.int32)
    col = edge_index[1].astype(jnp.int32)
    pad = E_PAD - N_EDGES
    pad_cols = N_NODES + 8 + (jnp.arange(pad, dtype=jnp.int32) % (N_PAD - N_NODES - 16))
    row_f = jnp.concatenate([row, jnp.zeros((pad,), jnp.int32)])
    col_f = jnp.concatenate([col, pad_cols])
    fill = NS * (CH0 - CH1) * CK
    fill_cols = N_NODES + 8 + (jnp.arange(fill, dtype=jnp.int32) % (N_PAD - N_NODES - 16))
    row_p = jnp.concatenate([
        row_f[:E_SC0].reshape(NS, CH0, CK),
        jnp.pad(row_f[E_SC0:].reshape(NS, CH1, CK), ((0, 0), (0, CH0 - CH1), (0, 0))),
    ])
    col_p = jnp.concatenate([
        col_f[:E_SC0].reshape(NS, CH0, CK),
        jnp.concatenate([col_f[E_SC0:].reshape(NS, CH1, CK),
                         fill_cols.reshape(NS, CH0 - CH1, CK)], axis=1),
    ])

    counts = _sc_count(col_p)
    c0 = counts[0, :N_NODES].reshape(N_NODES, 1)
    c1 = counts[1, :N_NODES].reshape(N_NODES, 1)

    y1, dis = _tc1(x, c0, c1, W1)
    p = _sc_agg(y1, row_p, col_p)
    y2 = _tc2(p[0, :N_NODES], p[1, :N_NODES], y1, dis, b1.reshape(1, D), W2)
    q = _sc_agg(y2, row_p, col_p)
    h, hs = _tc3(q[0, :N_NODES], q[1, :N_NODES], y2, dis, b2.reshape(1, D),
                 batch.astype(jnp.int32).reshape(N_NODES, 1))
    return (hs, h)


# speedup vs baseline: 1.3938x; 1.3938x over previous
import functools

import jax
import jax.numpy as jnp
from jax import lax
from jax.experimental import pallas as pl
from jax.experimental.pallas import tpu as pltpu
from jax.experimental.pallas import tpu_sc as plsc

N_NODES = 10000
N_PAD = 10240
N_EDGES = 320000
D = 128
N_GRAPHS = 128

NC = 2
NS = 16
NW = NC * NS
CH = 79
EPW = CH * 128
E_PAD = NW * EPW
RPT = N_PAD // NS

RB = 1000
GRID = N_NODES // RB

_mesh = plsc.VectorSubcoreMesh(core_axis_name="c", subcore_axis_name="s")


@functools.partial(
    pl.kernel,
    out_type=jax.ShapeDtypeStruct((NC, N_PAD), jnp.float32),
    mesh=_mesh,
    scratch_types=[
        pltpu.VMEM((CH, 128), jnp.int32),
        pltpu.VMEM((128,), jnp.float32),
        pltpu.VMEM((RPT,), jnp.float32),
        pltpu.VMEM_SHARED((N_PAD,), jnp.float32),
    ],
)
def _sc_count(col_hbm, out_hbm, col_v, ones_v, zbuf, acc):
    c = lax.axis_index("c")
    s = lax.axis_index("s")
    w = c * NS + s
    for l in range(8):
        ones_v[pl.ds(l * 16, 16)] = jnp.ones((16,), jnp.float32)

    def _zb(i, carry):
        zbuf[pl.ds(i * 16, 16)] = jnp.zeros((16,), jnp.float32)
        return carry

    lax.fori_loop(0, RPT // 16, _zb, 0)
    pltpu.sync_copy(zbuf, acc.at[pl.ds(s * RPT, RPT)])
    plsc.subcore_barrier()
    pltpu.sync_copy(col_hbm.at[w], col_v)

    def _body(j, carry):
        pltpu.sync_copy(ones_v, acc.at[col_v.at[j]], add=True)
        return carry

    lax.fori_loop(0, CH, _body, 0)
    plsc.subcore_barrier()
    pltpu.sync_copy(acc.at[pl.ds(s * RPT, RPT)],
                    out_hbm.at[c, pl.ds(s * RPT, RPT)])


@functools.partial(
    pl.kernel,
    out_type=jax.ShapeDtypeStruct((NC, N_PAD, D), jnp.float32),
    mesh=_mesh,
    scratch_types=[
        pltpu.VMEM((CH, 128), jnp.int32),
        pltpu.VMEM((CH, 128), jnp.int32),
        pltpu.VMEM((128, D), jnp.float32),
        pltpu.VMEM_SHARED((N_PAD, D), jnp.float32),
        pltpu.SemaphoreType.DMA,
    ],
)
def _sc_agg(y_hbm, row_hbm, col_hbm, out_hbm, row_v, col_v, gbuf, acc, sem):
    c = lax.axis_index("c")
    s = lax.axis_index("s")
    w = c * NS + s

    def _zr(i, carry):
        for l in range(8):
            gbuf[i, pl.ds(l * 16, 16)] = jnp.zeros((16,), jnp.float32)
        return carry

    lax.fori_loop(0, 128, _zr, 0)
    for k in range(RPT // 128):
        pltpu.sync_copy(gbuf, acc.at[pl.ds(s * RPT + k * 128, 128)])
    plsc.subcore_barrier()
    pltpu.sync_copy(row_hbm.at[w], row_v)
    pltpu.sync_copy(col_hbm.at[w], col_v)

    def _body(j, carry):
        pltpu.async_copy(y_hbm.at[row_v.at[j]], gbuf, sem).wait()
        pltpu.sync_copy(gbuf, acc.at[col_v.at[j]], add=True)
        return carry

    lax.fori_loop(0, CH, _body, 0)
    plsc.subcore_barrier()
    pltpu.sync_copy(acc.at[pl.ds(s * RPT, RPT)],
                    out_hbm.at[c, pl.ds(s * RPT, RPT)])


def _tc1_body(x_ref, c0_ref, c1_ref, w1_ref, y_ref, dis_ref):
    deg = c0_ref[...] + c1_ref[...] + 1.0
    dis = lax.rsqrt(deg)
    xw = jnp.dot(x_ref[...], w1_ref[...], preferred_element_type=jnp.float32)
    y_ref[...] = xw * dis
    dis_ref[...] = dis


def _tc2_body(p0_ref, p1_ref, y1_ref, dis_ref, b1_ref, w2_ref, y2_ref):
    dis = dis_ref[...]
    h1 = jnp.maximum(dis * (p0_ref[...] + p1_ref[...] + y1_ref[...]) + b1_ref[...], 0.0)
    y2_ref[...] = jnp.dot(h1, w2_ref[...], preferred_element_type=jnp.float32) * dis


def _tc3_body(q0_ref, q1_ref, y2_ref, dis_ref, b2_ref, batch_ref, h_ref, hs_ref):
    i = pl.program_id(0)
    h = dis_ref[...] * (q0_ref[...] + q1_ref[...] + y2_ref[...]) + b2_ref[...]
    h_ref[...] = h
    gids = lax.broadcasted_iota(jnp.int32, (RB, N_GRAPHS), 1)
    oh = (batch_ref[...] == gids).astype(jnp.float32)
    contrib = lax.dot_general(oh, h, (((0,), (0,)), ((), ())),
                              preferred_element_type=jnp.float32)

    @pl.when(i == 0)
    def _():
        hs_ref[...] = contrib

    @pl.when(i != 0)
    def _():
        hs_ref[...] += contrib


_row_spec = pl.BlockSpec((RB, D), lambda i: (i, 0))
_col1_spec = pl.BlockSpec((RB, 1), lambda i: (i, 0))
_w_spec = pl.BlockSpec((D, D), lambda i: (0, 0))
_b_spec = pl.BlockSpec((1, D), lambda i: (0, 0))

_tc1 = pl.pallas_call(
    _tc1_body,
    grid=(GRID,),
    in_specs=[_row_spec, _col1_spec, _col1_spec, _w_spec],
    out_specs=[_row_spec, _col1_spec],
    out_shape=[jax.ShapeDtypeStruct((N_NODES, D), jnp.float32),
               jax.ShapeDtypeStruct((N_NODES, 1), jnp.float32)],
)

_tc2 = pl.pallas_call(
    _tc2_body,
    grid=(GRID,),
    in_specs=[_row_spec, _row_spec, _row_spec, _col1_spec, _b_spec, _w_spec],
    out_specs=_row_spec,
    out_shape=jax.ShapeDtypeStruct((N_NODES, D), jnp.float32),
)

_tc3 = pl.pallas_call(
    _tc3_body,
    grid=(GRID,),
    in_specs=[_row_spec, _row_spec, _row_spec, _col1_spec, _b_spec, _col1_spec],
    out_specs=[_row_spec, pl.BlockSpec((N_GRAPHS, D), lambda i: (0, 0))],
    out_shape=[jax.ShapeDtypeStruct((N_NODES, D), jnp.float32),
               jax.ShapeDtypeStruct((N_GRAPHS, D), jnp.float32)],
)


def kernel(x, edge_index, batch, W1, b1, W2, b2):
    row = edge_index[0].astype(jnp.int32)
    col = edge_index[1].astype(jnp.int32)
    pad = E_PAD - N_EDGES
    row_p = jnp.concatenate([row, jnp.zeros((pad,), jnp.int32)]).reshape(NW, CH, 128)
    col_p = jnp.concatenate([col, jnp.full((pad,), N_NODES + 16, jnp.int32)]).reshape(NW, CH, 128)

    counts = _sc_count(col_p)
    c0 = counts[0, :N_NODES].reshape(N_NODES, 1)
    c1 = counts[1, :N_NODES].reshape(N_NODES, 1)

    y1, dis = _tc1(x, c0, c1, W1)
    p = _sc_agg(y1, row_p, col_p)
    y2 = _tc2(p[0, :N_NODES], p[1, :N_NODES], y1, dis, b1.reshape(1, D), W2)
    q = _sc_agg(y2, row_p, col_p)
    h, hs = _tc3(q[0, :N_NODES], q[1, :N_NODES], y2, dis, b2.reshape(1, D),
                 batch.astype(jnp.int32).reshape(N_NODES, 1))
    return (hs, h)
